# baseline (speedup 1.0000x reference)
import functools

import jax
import jax.numpy as jnp
from jax import lax
from jax.experimental import pallas as pl
from jax.experimental.pallas import tpu as pltpu
from jax.experimental.pallas import tpu_sc as plsc

N = 10000
Q = 128
E = 320000
DT = 0.1

NP = 10240
NROWS = NP // 128
NW = 32
CHUNK = 128
NCH = 160
SEC = 8
NSEC = NCH // SEC
CW = NCH * CHUNK
RPT = NP // 16
P1S = 5


def _sc_body(f_hbm, srcp_hbm, meta_hbm, t_out, wdeg_out,
             m4, rows, rows_bf, deg, cbuf, nodebuf, idx80,
             t_sh, deg_sh, w_sh,
             sem_g0, sem_g1, sem_s, sem_w0, sem_w1):
    cid = lax.axis_index("c")
    sid = lax.axis_index("s")
    wid = sid * 2 + cid
    zeros16f = jnp.zeros((16,), jnp.float32)
    ones16f = jnp.ones((16,), jnp.float32)
    iota16 = lax.iota(jnp.int32, 16)
    sem_g = (sem_g0, sem_g1)
    sem_w = (sem_w0, sem_w1)

    def zero_rows(i, carry):
        for g in range(8):
            rows[i, pl.ds(g * 16, 16)] = zeros16f
        return carry
    lax.fori_loop(0, CHUNK, zero_rows, 0)

    def zero_deg(i, carry):
        for g in range(8):
            deg[i, pl.ds(g * 16, 16)] = zeros16f
        return carry
    lax.fori_loop(0, NROWS, zero_deg, 0)

    def zero_cbuf(i, carry):
        cbuf[0, pl.ds(i * 16, 16)] = zeros16f
        return carry
    lax.fori_loop(0, CHUNK // 16, zero_cbuf, 0)

    def fill_idx(i, carry):
        idx80[pl.ds(i * 16, 16)] = iota16 + i * 16
        return carry
    lax.fori_loop(0, NROWS // 16, fill_idx, 0)

    def zero_acc(j, carry):
        pltpu.sync_copy(rows, t_sh.at[pl.ds(sid * RPT + j * 128, 128)])
        return carry
    lax.fori_loop(0, RPT // 128, zero_acc, 0)

    def zero_wsh(j, carry):
        pltpu.sync_copy(cbuf.at[0], w_sh.at[pl.ds((sid * 5 + j) * 128, 128)])
        return carry
    lax.fori_loop(0, RPT // 128, zero_wsh, 0)

    @pl.when(sid == 0)
    def _():
        pltpu.sync_copy(rows.at[pl.ds(0, NROWS)], deg_sh)

    def hist_sec(s, carry):
        pltpu.sync_copy(srcp_hbm.at[sid, s], m4)

        def hist_row(r, carry2):
            for q in range(4):
                for g in range(8):
                    v = m4[r, q, pl.ds(g * 16, 16)]
                    cnt, lastm = plsc.scan_count(v)
                    plsc.addupdate_scatter(
                        deg, [lax.shift_right_logical(v, 7), v & 127],
                        cnt.astype(jnp.float32), mask=lastm)
            return carry2
        lax.fori_loop(0, SEC, hist_row, 0)
        return carry
    lax.fori_loop(0, P1S, hist_sec, 0)

    plsc.subcore_barrier()
    pltpu.sync_copy(deg, deg_sh.at[idx80], add=True)
    plsc.subcore_barrier()
    pltpu.sync_copy(deg_sh, deg)

    def clamp_row(r, carry):
        for g in range(8):
            deg[r, pl.ds(g * 16, 16)] = jnp.maximum(
                deg[r, pl.ds(g * 16, 16)], ones16f)
        return carry
    lax.fori_loop(0, NROWS, clamp_row, 0)

    def step(s, jj, b):
        nb = 1 - b
        g = s * SEC + jj
        pltpu.make_async_copy(
            f_hbm.at[m4.at[jj, 1]], rows_bf.at[b], sem_g[b]).wait()

        @pl.when(jj < SEC - 1)
        def _():
            pltpu.async_copy(
                f_hbm.at[m4.at[jj + 1, 1]], rows_bf.at[nb], sem_g[nb])

        @pl.when(g >= 1)
        def _():
            pltpu.make_async_copy(
                rows, t_sh.at[nodebuf.at[b]], sem_s).wait()

        @pl.when(g >= 2)
        def _():
            pltpu.make_async_copy(
                cbuf.at[b], w_sh.at[nodebuf.at[b]], sem_w[b]).wait()

        for gq in range(8):
            nv = m4[jj, 2, pl.ds(gq * 16, 16)]
            wv = plsc.bitcast(m4[jj, 3, pl.ds(gq * 16, 16)], jnp.float32)
            d = plsc.load_gather(
                deg, [lax.shift_right_logical(nv, 7), nv & 127])
            cbuf[b, pl.ds(gq * 16, 16)] = wv / d
            nodebuf[b, pl.ds(gq * 16, 16)] = m4[jj, 0, pl.ds(gq * 16, 16)]

        @plsc.parallel_loop(0, CHUNK // 16, unroll=2)
        def _scale16(kk):
            c16 = cbuf[b, pl.ds(kk * 16, 16)]
            for l in range(16):
                cb = lax.broadcast_in_dim(c16[l], (16,), ())
                k = kk * 16 + l
                for g4 in range(4):
                    ri = rows_bf[b, k, pl.ds(g4 * 16, 16)]
                    rb = plsc.bitcast(ri, jnp.bfloat16)
                    av, bv = plsc.unpack(
                        rb, format=plsc.PackFormat.INTERLEAVED)
                    rows[k, pl.ds(g4 * 32, 16)] = jnp.maximum(av, 0.0) * cb
                    rows[k, pl.ds(g4 * 32 + 16, 16)] = (
                        jnp.maximum(bv, 0.0) * cb)

        pltpu.async_copy(rows, t_sh.at[nodebuf.at[b]], sem_s,
                         add=True)
        pltpu.async_copy(cbuf.at[b], w_sh.at[nodebuf.at[b]], sem_w[b],
                         add=True)

    def section_body(s, carry):
        pltpu.sync_copy(meta_hbm.at[wid, pl.ds(s * SEC, SEC)], m4)
        pltpu.async_copy(f_hbm.at[m4.at[0, 1]], rows_bf.at[0], sem_g[0])

        def pair_body(t, carry2):
            step(s, 2 * t, 0)
            step(s, 2 * t + 1, 1)
            return carry2
        lax.fori_loop(0, SEC // 2, pair_body, 0)
        return carry
    lax.fori_loop(0, NSEC, section_body, 0)

    pltpu.make_async_copy(rows, t_sh.at[nodebuf.at[0]], sem_s).wait()
    pltpu.make_async_copy(cbuf.at[0], w_sh.at[nodebuf.at[0]], sem_w[0]).wait()
    pltpu.make_async_copy(cbuf.at[1], w_sh.at[nodebuf.at[1]], sem_w[1]).wait()
    plsc.subcore_barrier()

    def writeout(j, carry):
        sl = pl.ds(sid * RPT + j * 128, 128)
        pltpu.sync_copy(t_sh.at[sl], t_out.at[cid, sl])
        return carry
    lax.fori_loop(0, RPT // 128, writeout, 0)
    pltpu.sync_copy(w_sh.at[pl.ds(sid * RPT, RPT)],
                    wdeg_out.at[cid, pl.ds(sid * RPT, RPT)])


_sc_mesh = plsc.VectorSubcoreMesh(core_axis_name="c", subcore_axis_name="s")

_sc_call = functools.partial(
    pl.kernel,
    out_type=(jax.ShapeDtypeStruct((2, NP, Q), jnp.float32),
              jax.ShapeDtypeStruct((2, NP), jnp.float32)),
    mesh=_sc_mesh,
    compiler_params=pltpu.CompilerParams(needs_layout_passes=False, use_tc_tiling_on_sc=False),
    scratch_types=[
        pltpu.VMEM((SEC, 4, CHUNK), jnp.int32),
        pltpu.VMEM((CHUNK, Q), jnp.float32),
        pltpu.VMEM((2, CHUNK, Q // 2), jnp.int32),
        pltpu.VMEM((NROWS, 128), jnp.float32),
        pltpu.VMEM((2, CHUNK), jnp.float32),
        pltpu.VMEM((2, CHUNK), jnp.int32),
        pltpu.VMEM((NROWS,), jnp.int32),
        pltpu.VMEM_SHARED((NP, Q), jnp.float32),
        pltpu.VMEM_SHARED((NROWS, 128), jnp.float32),
        pltpu.VMEM_SHARED((NP,), jnp.float32),
        pltpu.SemaphoreType.DMA,
        pltpu.SemaphoreType.DMA,
        pltpu.SemaphoreType.DMA,
        pltpu.SemaphoreType.DMA,
        pltpu.SemaphoreType.DMA,
    ],
)(_sc_body)


def _combine_body(xi_ref, f_ref, coll_ref, src_ref, t0_ref, t1_ref,
                  w0_ref, w1_ref, out_ref):
    fc = jnp.maximum(f_ref[...], 0.0)
    wdeg = w0_ref[...] + w1_ref[...]
    transport = xi_ref[...] * (t0_ref[...] + t1_ref[...] - wdeg * fc)
    out_ref[...] = jnp.maximum(
        fc - DT * (transport - coll_ref[...] - src_ref[...]), 0.0)


_row_spec = pl.BlockSpec((1000, Q), lambda i: (i, 0))
_col_spec = pl.BlockSpec((1000, 1), lambda i: (i, 0))
_combine_call = pl.pallas_call(
    _combine_body,
    grid=(N // 1000,),
    in_specs=[pl.BlockSpec((1, Q), lambda i: (0, 0)),
              _row_spec, _row_spec, _row_spec, _row_spec, _row_spec,
              _col_spec, _col_spec],
    out_specs=_row_spec,
    out_shape=jax.ShapeDtypeStruct((N, Q), jnp.float32),
)


def _make_colmap():
    m = [0] * Q
    for g in range(4):
        for i in range(16):
            m[32 * g + 2 * i] = 32 * g + i
            m[32 * g + 2 * i + 1] = 32 * g + 16 + i
    return tuple(m)


_COLMAP = _make_colmap()


def kernel(f_distribution, collision_term, source_term, edge_weight,
           xi_velocities, edge_index):
    src = edge_index[0]
    dst = edge_index[1]

    npad = NW * CW - 2 * E
    pad_node = (N + (jnp.arange(npad, dtype=jnp.int32) % (NP - N))).astype(
        jnp.int32)
    pad_other = (jnp.arange(npad, dtype=jnp.int32) % N).astype(jnp.int32)
    pad_nrm = jnp.full((npad,), NP - 1, jnp.int32)
    pad_w = jnp.zeros((npad,), jnp.int32)

    wbits = lax.bitcast_convert_type(edge_weight, jnp.int32)
    node2 = jnp.concatenate([src, dst, pad_node]).reshape(NW, NCH, CHUNK)
    other2 = jnp.concatenate([dst, src, pad_other]).reshape(NW, NCH, CHUNK)
    nrm2 = jnp.concatenate([src, src, pad_nrm]).reshape(NW, NCH, CHUNK)
    w2 = jnp.concatenate([wbits, wbits, pad_w]).reshape(NW, NCH, CHUNK)
    meta = jnp.stack([node2, other2, nrm2, w2], axis=2)

    epad = 16 * CW - E
    srcp = jnp.concatenate(
        [src, jnp.full((epad,), NP - 1, jnp.int32)]).reshape(
            16, P1S, SEC, 4, CHUNK)

    colmap = jnp.array(_COLMAP, dtype=jnp.int32)
    f_bf = jnp.take(f_distribution.astype(jnp.bfloat16), colmap, axis=1)
    f_i32 = lax.bitcast_convert_type(f_bf.reshape(N, Q // 2, 2), jnp.int32)
    t_part, w_part = _sc_call(f_i32, srcp, meta)

    return _combine_call(
        xi_velocities.reshape(1, Q), f_distribution, collision_term,
        source_term, t_part[0, :N], t_part[1, :N],
        w_part[0, :N].reshape(N, 1), w_part[1, :N].reshape(N, 1))


# speedup vs baseline: 1.4582x; 1.4582x over previous
import functools

import jax
import jax.numpy as jnp
from jax import lax
from jax.experimental import pallas as pl
from jax.experimental.pallas import tpu as pltpu
from jax.experimental.pallas import tpu_sc as plsc

N = 10000
Q = 128
E = 320000
DT = 0.1

NP = 10240
NROWS = NP // 128
NW = 32
CHUNK = 128
NCH = 160
SEC = 8
NSEC = NCH // SEC
CW = NCH * CHUNK
RPT = NP // 16
P1S = 5


def _sc_body(f_hbm, srcp_hbm, meta_hbm, t_out, wdeg_out,
             m4, rows2, deg, cbuf, nodebuf, idx80,
             t_sh, deg_sh, w_sh,
             sem_g0a, sem_g0b, sem_g1a, sem_g1b, sem_s0, sem_s1,
             sem_w0, sem_w1):
    cid = lax.axis_index("c")
    sid = lax.axis_index("s")
    wid = sid * 2 + cid
    zeros16f = jnp.zeros((16,), jnp.float32)
    ones16f = jnp.ones((16,), jnp.float32)
    iota16 = lax.iota(jnp.int32, 16)
    sem_ga = (sem_g0a, sem_g1a)
    sem_gb = (sem_g0b, sem_g1b)
    sem_s = (sem_s0, sem_s1)
    sem_w = (sem_w0, sem_w1)

    def zero_rows(i, carry):
        for g in range(8):
            rows2[0, i, pl.ds(g * 16, 16)] = zeros16f
        return carry
    lax.fori_loop(0, CHUNK, zero_rows, 0)

    def zero_deg(i, carry):
        for g in range(8):
            deg[i, pl.ds(g * 16, 16)] = zeros16f
        return carry
    lax.fori_loop(0, NROWS, zero_deg, 0)

    def zero_cbuf(i, carry):
        cbuf[0, pl.ds(i * 16, 16)] = zeros16f
        return carry
    lax.fori_loop(0, CHUNK // 16, zero_cbuf, 0)

    def fill_idx(i, carry):
        idx80[pl.ds(i * 16, 16)] = iota16 + i * 16
        return carry
    lax.fori_loop(0, NROWS // 16, fill_idx, 0)

    def zero_acc(j, carry):
        pltpu.sync_copy(rows2.at[0], t_sh.at[pl.ds(sid * RPT + j * 128, 128)])
        return carry
    lax.fori_loop(0, RPT // 128, zero_acc, 0)

    def zero_wsh(j, carry):
        pltpu.sync_copy(cbuf.at[0], w_sh.at[pl.ds((sid * 5 + j) * 128, 128)])
        return carry
    lax.fori_loop(0, RPT // 128, zero_wsh, 0)

    @pl.when(sid == 0)
    def _():
        pltpu.sync_copy(rows2.at[0, pl.ds(0, NROWS)], deg_sh)

    def hist_sec(s, carry):
        pltpu.sync_copy(srcp_hbm.at[sid, s], m4)

        def hist_row(r, carry2):
            for q in range(4):
                for g in range(8):
                    v = m4[r, q, pl.ds(g * 16, 16)]
                    cnt, lastm = plsc.scan_count(v)
                    plsc.addupdate_scatter(
                        deg, [lax.shift_right_logical(v, 7), v & 127],
                        cnt.astype(jnp.float32), mask=lastm)
            return carry2
        lax.fori_loop(0, SEC, hist_row, 0)
        return carry
    lax.fori_loop(0, P1S, hist_sec, 0)

    plsc.subcore_barrier()
    pltpu.sync_copy(deg, deg_sh.at[idx80], add=True)
    plsc.subcore_barrier()
    pltpu.sync_copy(deg_sh, deg)

    def clamp_row(r, carry):
        for g in range(8):
            deg[r, pl.ds(g * 16, 16)] = jnp.maximum(
                deg[r, pl.ds(g * 16, 16)], ones16f)
        return carry
    lax.fori_loop(0, NROWS, clamp_row, 0)

    def step(s, jj, b):
        nb = 1 - b
        g = s * SEC + jj
        pltpu.make_async_copy(
            f_hbm.at[m4.at[jj, 1, pl.ds(0, 64)]],
            rows2.at[b, pl.ds(0, 64)], sem_ga[b]).wait()
        pltpu.make_async_copy(
            f_hbm.at[m4.at[jj, 1, pl.ds(64, 64)]],
            rows2.at[b, pl.ds(64, 64)], sem_gb[b]).wait()

        @pl.when(jnp.logical_and(jj < SEC - 1, g >= 1))
        def _():
            pltpu.make_async_copy(
                rows2.at[nb], t_sh.at[nodebuf.at[nb]], sem_s[nb]).wait()

        @pl.when(jj < SEC - 1)
        def _():
            pltpu.async_copy(
                f_hbm.at[m4.at[jj + 1, 1, pl.ds(0, 64)]],
                rows2.at[nb, pl.ds(0, 64)], sem_ga[nb])
            pltpu.async_copy(
                f_hbm.at[m4.at[jj + 1, 1, pl.ds(64, 64)]],
                rows2.at[nb, pl.ds(64, 64)], sem_gb[nb])

        @pl.when(g >= 2)
        def _():
            pltpu.make_async_copy(
                cbuf.at[b], w_sh.at[nodebuf.at[b]], sem_w[b]).wait()

        for gq in range(8):
            nv = m4[jj, 2, pl.ds(gq * 16, 16)]
            wv = plsc.bitcast(m4[jj, 3, pl.ds(gq * 16, 16)], jnp.float32)
            d = plsc.load_gather(
                deg, [lax.shift_right_logical(nv, 7), nv & 127])
            cbuf[b, pl.ds(gq * 16, 16)] = wv / d
            nodebuf[b, pl.ds(gq * 16, 16)] = m4[jj, 0, pl.ds(gq * 16, 16)]

        @plsc.parallel_loop(0, CHUNK // 16, unroll=2)
        def _scale16(kk):
            c16 = cbuf[b, pl.ds(kk * 16, 16)]
            for l in range(16):
                cb = lax.broadcast_in_dim(c16[l], (16,), ())
                k = kk * 16 + l
                for gq in range(8):
                    rows2[b, k, pl.ds(gq * 16, 16)] = jnp.maximum(
                        rows2[b, k, pl.ds(gq * 16, 16)], 0.0) * cb

        pltpu.async_copy(rows2.at[b], t_sh.at[nodebuf.at[b]], sem_s[b],
                         add=True)
        pltpu.async_copy(cbuf.at[b], w_sh.at[nodebuf.at[b]], sem_w[b],
                         add=True)

    def section_body(s, carry):
        pltpu.sync_copy(meta_hbm.at[wid, pl.ds(s * SEC, SEC)], m4)

        @pl.when(s >= 1)
        def _():
            pltpu.make_async_copy(
                rows2.at[0], t_sh.at[nodebuf.at[0]], sem_s[0]).wait()
        pltpu.async_copy(f_hbm.at[m4.at[0, 1, pl.ds(0, 64)]],
                         rows2.at[0, pl.ds(0, 64)], sem_ga[0])
        pltpu.async_copy(f_hbm.at[m4.at[0, 1, pl.ds(64, 64)]],
                         rows2.at[0, pl.ds(64, 64)], sem_gb[0])

        def pair_body(t, carry2):
            step(s, 2 * t, 0)
            step(s, 2 * t + 1, 1)
            return carry2
        lax.fori_loop(0, SEC // 2, pair_body, 0)
        return carry
    lax.fori_loop(0, NSEC, section_body, 0)

    pltpu.make_async_copy(rows2.at[0], t_sh.at[nodebuf.at[0]], sem_s[0]).wait()
    pltpu.make_async_copy(rows2.at[1], t_sh.at[nodebuf.at[1]], sem_s[1]).wait()
    pltpu.make_async_copy(cbuf.at[0], w_sh.at[nodebuf.at[0]], sem_w[0]).wait()
    pltpu.make_async_copy(cbuf.at[1], w_sh.at[nodebuf.at[1]], sem_w[1]).wait()
    plsc.subcore_barrier()

    def writeout(j, carry):
        sl = pl.ds(sid * RPT + j * 128, 128)
        pltpu.sync_copy(t_sh.at[sl], t_out.at[cid, sl])
        return carry
    lax.fori_loop(0, RPT // 128, writeout, 0)
    pltpu.sync_copy(w_sh.at[pl.ds(sid * RPT, RPT)],
                    wdeg_out.at[cid, pl.ds(sid * RPT, RPT)])


_sc_mesh = plsc.VectorSubcoreMesh(core_axis_name="c", subcore_axis_name="s")

_sc_call = functools.partial(
    pl.kernel,
    out_type=(jax.ShapeDtypeStruct((2, NP, Q), jnp.float32),
              jax.ShapeDtypeStruct((2, NP), jnp.float32)),
    mesh=_sc_mesh,
    compiler_params=pltpu.CompilerParams(needs_layout_passes=False),
    scratch_types=[
        pltpu.VMEM((SEC, 4, CHUNK), jnp.int32),
        pltpu.VMEM((2, CHUNK, Q), jnp.float32),
        pltpu.VMEM((NROWS, 128), jnp.float32),
        pltpu.VMEM((2, CHUNK), jnp.float32),
        pltpu.VMEM((2, CHUNK), jnp.int32),
        pltpu.VMEM((NROWS,), jnp.int32),
        pltpu.VMEM_SHARED((NP, Q), jnp.float32),
        pltpu.VMEM_SHARED((NROWS, 128), jnp.float32),
        pltpu.VMEM_SHARED((NP,), jnp.float32),
        pltpu.SemaphoreType.DMA,
        pltpu.SemaphoreType.DMA,
        pltpu.SemaphoreType.DMA,
        pltpu.SemaphoreType.DMA,
        pltpu.SemaphoreType.DMA,
        pltpu.SemaphoreType.DMA,
        pltpu.SemaphoreType.DMA,
        pltpu.SemaphoreType.DMA,
    ],
)(_sc_body)


def _combine_body(xi_ref, f_ref, coll_ref, src_ref, t0_ref, t1_ref,
                  w0_ref, w1_ref, out_ref):
    fc = jnp.maximum(f_ref[...], 0.0)
    wdeg = w0_ref[...] + w1_ref[...]
    transport = xi_ref[...] * (t0_ref[...] + t1_ref[...] - wdeg * fc)
    out_ref[...] = jnp.maximum(
        fc - DT * (transport - coll_ref[...] - src_ref[...]), 0.0)


_row_spec = pl.BlockSpec((1000, Q), lambda i: (i, 0))
_col_spec = pl.BlockSpec((1000, 1), lambda i: (i, 0))
_combine_call = pl.pallas_call(
    _combine_body,
    grid=(N // 1000,),
    in_specs=[pl.BlockSpec((1, Q), lambda i: (0, 0)),
              _row_spec, _row_spec, _row_spec, _row_spec, _row_spec,
              _col_spec, _col_spec],
    out_specs=_row_spec,
    out_shape=jax.ShapeDtypeStruct((N, Q), jnp.float32),
)


def kernel(f_distribution, collision_term, source_term, edge_weight,
           xi_velocities, edge_index):
    src = edge_index[0]
    dst = edge_index[1]

    npad = NW * CW - 2 * E
    pad_node = (N + (jnp.arange(npad, dtype=jnp.int32) % (NP - N))).astype(
        jnp.int32)
    pad_other = (jnp.arange(npad, dtype=jnp.int32) % N).astype(jnp.int32)
    pad_nrm = jnp.full((npad,), NP - 1, jnp.int32)
    pad_w = jnp.zeros((npad,), jnp.int32)

    wbits = lax.bitcast_convert_type(edge_weight, jnp.int32)
    node2 = jnp.concatenate([src, dst, pad_node]).reshape(NW, NCH, CHUNK)
    other2 = jnp.concatenate([dst, src, pad_other]).reshape(NW, NCH, CHUNK)
    nrm2 = jnp.concatenate([src, src, pad_nrm]).reshape(NW, NCH, CHUNK)
    w2 = jnp.concatenate([wbits, wbits, pad_w]).reshape(NW, NCH, CHUNK)
    meta = jnp.stack([node2, other2, nrm2, w2], axis=2)

    epad = 16 * CW - E
    srcp = jnp.concatenate(
        [src, jnp.full((epad,), NP - 1, jnp.int32)]).reshape(
            16, P1S, SEC, 4, CHUNK)

    t_part, w_part = _sc_call(f_distribution, srcp, meta)

    return _combine_call(
        xi_velocities.reshape(1, Q), f_distribution, collision_term,
        source_term, t_part[0, :N], t_part[1, :N],
        w_part[0, :N].reshape(N, 1), w_part[1, :N].reshape(N, 1))
